# baseline (speedup 1.0000x reference)
import jax
import jax.numpy as jnp
from jax.experimental import pallas as pl
from jax.experimental.pallas import tpu as pltpu


def _copy_body(emb_ref, out_ref, sem):
    pltpu.make_async_copy(emb_ref, out_ref.at[0], sem).start()
    pltpu.make_async_copy(emb_ref, out_ref.at[0], sem).wait()


def kernel(x, embedding):
    seq_len = x.shape[1]
    max_len, d_model = embedding.shape
    out = pl.pallas_call(
        _copy_body,
        in_specs=[pl.BlockSpec(memory_space=pl.ANY)],
        out_specs=pl.BlockSpec(memory_space=pl.ANY),
        out_shape=jax.ShapeDtypeStruct((1, seq_len, d_model), jnp.float32),
        scratch_shapes=[pltpu.SemaphoreType.DMA],
    )(embedding)
    return out


# speedup vs baseline: 20.5903x; 20.5903x over previous
import functools

import jax
import jax.numpy as jnp
from jax import lax
from jax.experimental import pallas as pl
from jax.experimental.pallas import tpu as pltpu
from jax.experimental.pallas import tpu_sc as plsc

_NC, _NS = 2, 16
_NW = _NC * _NS


def _make_sc_copy(max_len, d_model, nbuf, chunk):
    rows_per_w = max_len // _NW
    nchunk = rows_per_w // chunk
    mesh = plsc.VectorSubcoreMesh(core_axis_name="c", subcore_axis_name="s")
    scratch = [pltpu.VMEM((chunk, d_model), jnp.float32) for _ in range(nbuf)]
    scratch += [pltpu.SemaphoreType.DMA for _ in range(2 * nbuf)]

    @functools.partial(
        pl.kernel,
        out_type=jax.ShapeDtypeStruct((max_len, d_model), jnp.float32),
        mesh=mesh,
        scratch_types=scratch,
    )
    def sc_copy(emb_hbm, out_hbm, *scr):
        bufs = scr[:nbuf]
        in_sems = scr[nbuf:2 * nbuf]
        out_sems = scr[2 * nbuf:]
        wid = lax.axis_index("s") * _NC + lax.axis_index("c")
        base = wid * rows_per_w
        in_cp = [None] * nchunk
        out_cp = [None] * nchunk
        for j in range(min(nbuf, nchunk)):
            in_cp[j] = pltpu.async_copy(
                emb_hbm.at[pl.ds(base + j * chunk, chunk)], bufs[j], in_sems[j])
        for k in range(nchunk):
            b = k % nbuf
            in_cp[k].wait()
            out_cp[k] = pltpu.async_copy(
                bufs[b], out_hbm.at[pl.ds(base + k * chunk, chunk)], out_sems[b])
            nk = k + nbuf
            if nk < nchunk:
                out_cp[k].wait()
                in_cp[nk] = pltpu.async_copy(
                    emb_hbm.at[pl.ds(base + nk * chunk, chunk)], bufs[b],
                    in_sems[b])
        for k in range(max(0, nchunk - nbuf), nchunk):
            out_cp[k].wait()

    return sc_copy


def kernel(x, embedding):
    seq_len = x.shape[1]
    max_len, d_model = embedding.shape
    sc_copy = _make_sc_copy(max_len, d_model, nbuf=2, chunk=64)
    out = sc_copy(embedding)
    return out[None, :seq_len, :]
